# baseline (device time: 115442 ns/iter reference)
import math

import jax
import jax.numpy as jnp
from jax import lax
from jax.experimental import pallas as pl
from jax.experimental.pallas import tpu as pltpu

N_DEV = 4
BLK_Q = 512

P1R_KT, P1R_VT, P1R_KB, P1R_VB = 0, 1, 2, 3
P1L_KT, P1L_VT, P1L_KB, P1L_VB = 4, 5, 6, 7
P2R_K, P2R_V = 8, 9
P2L_K, P2L_V = 10, 11


def kernel(q, k, v):
    s_loc, d = q.shape
    scale = math.log2(math.e) / (d ** 0.5)
    n_blk = s_loc // BLK_Q
    half = s_loc // 2

    def body(q_ref, k_ref, v_ref, out_ref, qs_ref, kg_ref, vg_ref,
             l_ref, acc_ref, send_sems, recv_sems):
        my = lax.axis_index("i")
        left = lax.rem(my + N_DEV - 1, N_DEV)
        right = lax.rem(my + 1, N_DEV)

        top = pl.ds(0, half)
        bot = pl.ds(half, half)

        def copy(src, dst, sem_idx, dev):
            return pltpu.make_async_remote_copy(
                src_ref=src,
                dst_ref=dst,
                send_sem=send_sems.at[sem_idx],
                recv_sem=recv_sems.at[sem_idx],
                device_id=(dev,),
                device_id_type=pl.DeviceIdType.MESH,
            )

        barrier = pltpu.get_barrier_semaphore()
        for nbr in (left, right):
            pl.semaphore_signal(
                barrier, inc=1,
                device_id=(nbr,), device_id_type=pl.DeviceIdType.MESH,
            )
        pl.semaphore_wait(barrier, 2)

        kg_ref[0] = k_ref[...].astype(jnp.bfloat16)
        vg_ref[0] = v_ref[...].astype(jnp.bfloat16)

        p1 = [
            copy(kg_ref.at[0, top], kg_ref.at[1, top], P1R_KT, right),
            copy(kg_ref.at[0, top], kg_ref.at[2, top], P1L_KT, left),
            copy(vg_ref.at[0, top], vg_ref.at[1, top], P1R_VT, right),
            copy(vg_ref.at[0, top], vg_ref.at[2, top], P1L_VT, left),
            copy(kg_ref.at[0, bot], kg_ref.at[1, bot], P1R_KB, right),
            copy(kg_ref.at[0, bot], kg_ref.at[2, bot], P1L_KB, left),
            copy(vg_ref.at[0, bot], vg_ref.at[1, bot], P1R_VB, right),
            copy(vg_ref.at[0, bot], vg_ref.at[2, bot], P1L_VB, left),
        ]
        for r in p1:
            r.start()

        qs_ref[...] = (q_ref[...] * scale).astype(jnp.bfloat16)

        def contrib(parts, first=False, last=False):
            def compute_block(b, carry):
                ds = pl.ds(b * BLK_Q, BLK_Q)
                qb = qs_ref[ds, :]
                l_c = None
                o_c = None
                for c, hs in parts:
                    kc = kg_ref[c] if hs is None else kg_ref[c, hs, :]
                    vc = vg_ref[c] if hs is None else vg_ref[c, hs, :]
                    s = lax.dot_general(
                        qb, kc,
                        dimension_numbers=(((1,), (1,)), ((), ())),
                        preferred_element_type=jnp.float32,
                    )
                    w = jnp.exp2(s).astype(jnp.bfloat16)
                    l_p = jnp.sum(w, axis=1, keepdims=True,
                                  dtype=jnp.float32)
                    o_p = lax.dot_general(
                        w, vc,
                        dimension_numbers=(((1,), (0,)), ((), ())),
                        preferred_element_type=jnp.float32,
                    )
                    l_c = l_p if l_c is None else l_c + l_p
                    o_c = o_p if o_c is None else o_c + o_p
                if first:
                    l_ref[ds, :] = l_c
                    acc_ref[ds, :] = o_c
                elif last:
                    out_ref[ds, :] = (acc_ref[ds, :] + o_c) / (
                        l_ref[ds, :] + l_c
                    )
                else:
                    l_ref[ds, :] += l_c
                    acc_ref[ds, :] += o_c
                return carry

            lax.fori_loop(0, n_blk, compute_block, 0)

        contrib([(0, None)], first=True)

        p1[0].wait_recv()
        p1[2].wait_recv()
        p2_r = [
            copy(kg_ref.at[1, top], kg_ref.at[3, top], P2R_K, right),
            copy(vg_ref.at[1, top], vg_ref.at[3, top], P2R_V, right),
        ]
        for r in p2_r:
            r.start()
        contrib([(1, top)])

        p1[1].wait_recv()
        p1[3].wait_recv()
        contrib([(2, top)])

        p1[4].wait_recv()
        p1[6].wait_recv()
        p1[5].wait_recv()
        p1[7].wait_recv()
        p2_l = [
            copy(kg_ref.at[2, bot], kg_ref.at[3, bot], P2L_K, left),
            copy(vg_ref.at[2, bot], vg_ref.at[3, bot], P2L_V, left),
        ]
        for r in p2_l:
            r.start()
        contrib([(1, bot), (2, bot)])

        for r in p2_r + p2_l:
            r.wait_recv()
        contrib([(3, None)], last=True)

        for r in p1 + p2_r + p2_l:
            r.wait_send()

    return pl.pallas_call(
        body,
        out_shape=jax.ShapeDtypeStruct((s_loc, d), jnp.float32),
        in_specs=[pl.BlockSpec(memory_space=pltpu.VMEM)] * 3,
        out_specs=pl.BlockSpec(memory_space=pltpu.VMEM),
        scratch_shapes=[
            pltpu.VMEM((s_loc, d), jnp.bfloat16),
            pltpu.VMEM((N_DEV, s_loc, d), jnp.bfloat16),
            pltpu.VMEM((N_DEV, s_loc, d), jnp.bfloat16),
            pltpu.VMEM((s_loc, 1), jnp.float32),
            pltpu.VMEM((s_loc, d), jnp.float32),
            pltpu.SemaphoreType.DMA((12,)),
            pltpu.SemaphoreType.DMA((12,)),
        ],
        compiler_params=pltpu.CompilerParams(
            collective_id=0,
            vmem_limit_bytes=100 * 1024 * 1024,
        ),
    )(q, k, v)


# device time: 109624 ns/iter; 1.0531x vs baseline; 1.0531x over previous
import math

import jax
import jax.numpy as jnp
from jax import lax
from jax.experimental import pallas as pl
from jax.experimental.pallas import tpu as pltpu

N_DEV = 4
BLK_Q = 2048

P1R_KT, P1R_VT, P1R_KB, P1R_VB = 0, 1, 2, 3
P1L_KT, P1L_VT, P1L_KB, P1L_VB = 4, 5, 6, 7
P2R_K, P2R_V = 8, 9
P2L_K, P2L_V = 10, 11


def kernel(q, k, v):
    s_loc, d = q.shape
    scale = math.log2(math.e) / (d ** 0.5)
    n_blk = s_loc // BLK_Q
    half = s_loc // 2

    def body(q_ref, k_ref, v_ref, out_ref, qs_ref, kg_ref, vg_ref,
             l_ref, acc_ref, send_sems, recv_sems):
        my = lax.axis_index("i")
        left = lax.rem(my + N_DEV - 1, N_DEV)
        right = lax.rem(my + 1, N_DEV)

        top = pl.ds(0, half)
        bot = pl.ds(half, half)

        def copy(src, dst, sem_idx, dev):
            return pltpu.make_async_remote_copy(
                src_ref=src,
                dst_ref=dst,
                send_sem=send_sems.at[sem_idx],
                recv_sem=recv_sems.at[sem_idx],
                device_id=(dev,),
                device_id_type=pl.DeviceIdType.MESH,
            )

        barrier = pltpu.get_barrier_semaphore()
        for nbr in (left, right):
            pl.semaphore_signal(
                barrier, inc=1,
                device_id=(nbr,), device_id_type=pl.DeviceIdType.MESH,
            )
        pl.semaphore_wait(barrier, 2)

        kg_ref[0] = k_ref[...].astype(jnp.bfloat16)
        vg_ref[0] = v_ref[...].astype(jnp.bfloat16)

        p1 = [
            copy(kg_ref.at[0, top], kg_ref.at[1, top], P1R_KT, right),
            copy(kg_ref.at[0, top], kg_ref.at[2, top], P1L_KT, left),
            copy(vg_ref.at[0, top], vg_ref.at[1, top], P1R_VT, right),
            copy(vg_ref.at[0, top], vg_ref.at[2, top], P1L_VT, left),
            copy(kg_ref.at[0, bot], kg_ref.at[1, bot], P1R_KB, right),
            copy(kg_ref.at[0, bot], kg_ref.at[2, bot], P1L_KB, left),
            copy(vg_ref.at[0, bot], vg_ref.at[1, bot], P1R_VB, right),
            copy(vg_ref.at[0, bot], vg_ref.at[2, bot], P1L_VB, left),
        ]
        for r in p1:
            r.start()

        qs_ref[...] = (q_ref[...] * scale).astype(jnp.bfloat16)

        def contrib(parts, first=False, last=False):
            def compute_block(b, carry):
                ds = pl.ds(b * BLK_Q, BLK_Q)
                qb = qs_ref[ds, :]
                l_c = None
                o_c = None
                for c, hs in parts:
                    kc = kg_ref[c] if hs is None else kg_ref[c, hs, :]
                    vc = vg_ref[c] if hs is None else vg_ref[c, hs, :]
                    s = lax.dot_general(
                        qb, kc,
                        dimension_numbers=(((1,), (1,)), ((), ())),
                        preferred_element_type=jnp.float32,
                    )
                    w = jnp.exp2(s).astype(jnp.bfloat16)
                    l_p = jnp.sum(w, axis=1, keepdims=True,
                                  dtype=jnp.float32)
                    o_p = lax.dot_general(
                        w, vc,
                        dimension_numbers=(((1,), (0,)), ((), ())),
                        preferred_element_type=jnp.float32,
                    )
                    l_c = l_p if l_c is None else l_c + l_p
                    o_c = o_p if o_c is None else o_c + o_p
                if first:
                    l_ref[ds, :] = l_c
                    acc_ref[ds, :] = o_c
                elif last:
                    out_ref[ds, :] = (acc_ref[ds, :] + o_c) / (
                        l_ref[ds, :] + l_c
                    )
                else:
                    l_ref[ds, :] += l_c
                    acc_ref[ds, :] += o_c
                return carry

            lax.fori_loop(0, n_blk, compute_block, 0)

        contrib([(0, top)], first=True)
        contrib([(0, bot)])

        p1[0].wait_recv()
        p1[2].wait_recv()
        p2_r = [
            copy(kg_ref.at[1, top], kg_ref.at[3, top], P2R_K, right),
            copy(vg_ref.at[1, top], vg_ref.at[3, top], P2R_V, right),
        ]
        for r in p2_r:
            r.start()
        contrib([(1, top)])

        p1[1].wait_recv()
        p1[3].wait_recv()
        contrib([(2, top)])

        p1[4].wait_recv()
        p1[6].wait_recv()
        contrib([(1, bot)])

        p1[5].wait_recv()
        p1[7].wait_recv()
        p2_l = [
            copy(kg_ref.at[2, bot], kg_ref.at[3, bot], P2L_K, left),
            copy(vg_ref.at[2, bot], vg_ref.at[3, bot], P2L_V, left),
        ]
        for r in p2_l:
            r.start()
        contrib([(2, bot)])

        for r in p2_r:
            r.wait_recv()
        contrib([(3, top)])
        for r in p2_l:
            r.wait_recv()
        contrib([(3, bot)], last=True)

        for r in p1 + p2_r + p2_l:
            r.wait_send()

    return pl.pallas_call(
        body,
        out_shape=jax.ShapeDtypeStruct((s_loc, d), jnp.float32),
        in_specs=[pl.BlockSpec(memory_space=pltpu.VMEM)] * 3,
        out_specs=pl.BlockSpec(memory_space=pltpu.VMEM),
        scratch_shapes=[
            pltpu.VMEM((s_loc, d), jnp.bfloat16),
            pltpu.VMEM((N_DEV, s_loc, d), jnp.bfloat16),
            pltpu.VMEM((N_DEV, s_loc, d), jnp.bfloat16),
            pltpu.VMEM((s_loc, 1), jnp.float32),
            pltpu.VMEM((s_loc, d), jnp.float32),
            pltpu.SemaphoreType.DMA((12,)),
            pltpu.SemaphoreType.DMA((12,)),
        ],
        compiler_params=pltpu.CompilerParams(
            collective_id=0,
            vmem_limit_bytes=63 * 1024 * 1024,
        ),
    )(q, k, v)


# device time: 108113 ns/iter; 1.0678x vs baseline; 1.0140x over previous
import math

import jax
import jax.numpy as jnp
from jax import lax
from jax.experimental import pallas as pl
from jax.experimental.pallas import tpu as pltpu

N_DEV = 4
BLK_Q = 2048

P1R_KT, P1R_VT, P1R_KB, P1R_VB = 0, 1, 2, 3
P1L_KT, P1L_VT, P1L_KB, P1L_VB = 4, 5, 6, 7
P2R_K, P2R_V = 8, 9
P2L_K, P2L_V = 10, 11


def kernel(q, k, v):
    s_loc, d = q.shape
    scale = math.log2(math.e) / (d ** 0.5)
    n_blk = s_loc // BLK_Q
    half = s_loc // 2

    def body(q_ref, k_ref, v_ref, out_ref, qs_ref, kg_ref, vg_ref,
             l_ref, acc_ref, send_sems, recv_sems):
        my = lax.axis_index("i")
        left = lax.rem(my + N_DEV - 1, N_DEV)
        right = lax.rem(my + 1, N_DEV)

        top = pl.ds(0, half)
        bot = pl.ds(half, half)

        def copy(src, dst, sem_idx, dev):
            return pltpu.make_async_remote_copy(
                src_ref=src,
                dst_ref=dst,
                send_sem=send_sems.at[sem_idx],
                recv_sem=recv_sems.at[sem_idx],
                device_id=(dev,),
                device_id_type=pl.DeviceIdType.MESH,
            )

        barrier = pltpu.get_barrier_semaphore()
        for nbr in (left, right):
            pl.semaphore_signal(
                barrier, inc=1,
                device_id=(nbr,), device_id_type=pl.DeviceIdType.MESH,
            )
        pl.semaphore_wait(barrier, 2)

        kg_ref[0] = k_ref[...].astype(jnp.bfloat16)
        vg_ref[0] = v_ref[...].astype(jnp.bfloat16)

        p1 = [
            copy(kg_ref.at[0, top], kg_ref.at[1, top], P1R_KT, right),
            copy(kg_ref.at[0, top], kg_ref.at[2, top], P1L_KT, left),
            copy(vg_ref.at[0, top], vg_ref.at[1, top], P1R_VT, right),
            copy(vg_ref.at[0, top], vg_ref.at[2, top], P1L_VT, left),
            copy(kg_ref.at[0, bot], kg_ref.at[1, bot], P1R_KB, right),
            copy(kg_ref.at[0, bot], kg_ref.at[2, bot], P1L_KB, left),
            copy(vg_ref.at[0, bot], vg_ref.at[1, bot], P1R_VB, right),
            copy(vg_ref.at[0, bot], vg_ref.at[2, bot], P1L_VB, left),
        ]
        for r in p1:
            r.start()

        qs_ref[...] = (q_ref[...] * scale).astype(jnp.bfloat16)

        def contrib(parts, first=False, last=False):
            def compute_block(b, carry):
                ds = pl.ds(b * BLK_Q, BLK_Q)
                qb = qs_ref[ds, :]
                l_c = None
                o_c = None
                for c, hs in parts:
                    kc = kg_ref[c] if hs is None else kg_ref[c, hs, :]
                    vc = vg_ref[c] if hs is None else vg_ref[c, hs, :]
                    s = lax.dot_general(
                        qb, kc,
                        dimension_numbers=(((1,), (1,)), ((), ())),
                        preferred_element_type=jnp.float32,
                    )
                    w = jnp.exp2(s).astype(jnp.bfloat16)
                    l_p = jnp.sum(w, axis=1, keepdims=True,
                                  dtype=jnp.float32)
                    o_p = lax.dot_general(
                        w, vc,
                        dimension_numbers=(((1,), (0,)), ((), ())),
                        preferred_element_type=jnp.float32,
                    )
                    l_c = l_p if l_c is None else l_c + l_p
                    o_c = o_p if o_c is None else o_c + o_p
                if first:
                    l_ref[ds, :] = l_c
                    acc_ref[ds, :] = o_c
                elif last:
                    out_ref[ds, :] = (acc_ref[ds, :] + o_c) / (
                        l_ref[ds, :] + l_c
                    )
                else:
                    l_ref[ds, :] += l_c
                    acc_ref[ds, :] += o_c
                return carry

            lax.fori_loop(0, n_blk, compute_block, 0)

        contrib([(0, top)], first=True)
        contrib([(0, bot)])

        p1[0].wait_recv()
        p1[2].wait_recv()
        p2_r = [
            copy(kg_ref.at[1, top], kg_ref.at[3, top], P2R_K, right),
            copy(vg_ref.at[1, top], vg_ref.at[3, top], P2R_V, right),
        ]
        for r in p2_r:
            r.start()
        contrib([(1, top)])

        p1[1].wait_recv()
        p1[3].wait_recv()
        contrib([(2, top)])

        p1[4].wait_recv()
        p1[6].wait_recv()
        p1[5].wait_recv()
        p1[7].wait_recv()
        p2_l = [
            copy(kg_ref.at[2, bot], kg_ref.at[3, bot], P2L_K, left),
            copy(vg_ref.at[2, bot], vg_ref.at[3, bot], P2L_V, left),
        ]
        for r in p2_l:
            r.start()
        contrib([(1, bot)])
        contrib([(2, bot)])

        for r in p2_r:
            r.wait_recv()
        contrib([(3, top)])
        for r in p2_l:
            r.wait_recv()
        contrib([(3, bot)], last=True)

        for r in p1 + p2_r + p2_l:
            r.wait_send()

    return pl.pallas_call(
        body,
        out_shape=jax.ShapeDtypeStruct((s_loc, d), jnp.float32),
        in_specs=[pl.BlockSpec(memory_space=pltpu.VMEM)] * 3,
        out_specs=pl.BlockSpec(memory_space=pltpu.VMEM),
        scratch_shapes=[
            pltpu.VMEM((s_loc, d), jnp.bfloat16),
            pltpu.VMEM((N_DEV, s_loc, d), jnp.bfloat16),
            pltpu.VMEM((N_DEV, s_loc, d), jnp.bfloat16),
            pltpu.VMEM((s_loc, 1), jnp.float32),
            pltpu.VMEM((s_loc, d), jnp.float32),
            pltpu.SemaphoreType.DMA((12,)),
            pltpu.SemaphoreType.DMA((12,)),
        ],
        compiler_params=pltpu.CompilerParams(
            collective_id=0,
            vmem_limit_bytes=63 * 1024 * 1024,
        ),
    )(q, k, v)
